# baseline (speedup 1.0000x reference)
import jax
import jax.numpy as jnp
from jax import lax
from jax.experimental import pallas as pl
from jax.experimental.pallas import tpu as pltpu
from jax.experimental.pallas import tpu_sc as plsc

_N_STATES = 65
_N_OBVS = 100000
_N_LOOKUPS = 3276800

_NC = 2
_NS = 16
_NW = _NC * _NS
_LANES = 16

_C = 1024
_G = 1024
_PW = _N_LOOKUPS // _NW
_NCHUNK = _PW // _C
_NG = _N_LOOKUPS // _C


_NBUF = 8
_GD = _NBUF - 1


def _sc_gather_body(flat_hbm, state_hbm, obs_hbm, out_hbm, *refs):
    n = _NBUF
    sv = refs[0 * n:1 * n]
    ov = refs[1 * n:2 * n]
    ix = refs[2 * n:3 * n]
    ot = refs[3 * n:4 * n]
    isem = refs[4 * n:5 * n]
    gsem = refs[5 * n:6 * n]
    osem = refs[6 * n:7 * n]

    cid = lax.axis_index("c")
    sid = lax.axis_index("s")
    wid = sid * _NC + cid
    base = wid * _PW

    def in_copies(b, c):
        off = base + c * _C
        return (pltpu.make_async_copy(state_hbm.at[pl.ds(off, _C)], sv[b],
                                      isem[b]),
                pltpu.make_async_copy(obs_hbm.at[pl.ds(off, _C)], ov[b],
                                      isem[b]))

    def issue_in(b, c):
        for cp in in_copies(b, c):
            cp.start()

    def wait_in(b, c):
        for cp in in_copies(b, c):
            cp.wait()

    def compute(b):
        def vec_body(j, carry):
            st = sv[b][pl.ds(j * _LANES, _LANES)]
            ob = ov[b][pl.ds(j * _LANES, _LANES)]
            ix[b][pl.ds(j * _LANES, _LANES)] = st * _N_OBVS + ob
            return carry
        lax.fori_loop(0, _C // _LANES, vec_body, 0, unroll=4)

    def gather_copies(b):
        return [pltpu.make_async_copy(flat_hbm.at[ix[b].at[pl.ds(k * _G, _G)]],
                                      ot[b].at[pl.ds(k * _G, _G)], gsem[b])
                for k in range(_C // _G)]

    def fire_gather(b):
        for cp in gather_copies(b):
            cp.start()

    def wait_gather(b):
        for cp in gather_copies(b):
            cp.wait()

    def store_copy(b, c):
        return pltpu.make_async_copy(ot[b], out_hbm.at[pl.ds(base + c * _C, _C)],
                                     osem[b])

    def fire_store(b, c):
        store_copy(b, c).start()

    def wait_store(b, c):
        store_copy(b, c).wait()

    def stage(c, b, drain=True, wstore=True, issue=True):
        wait_in(b, c)
        compute(b)
        if wstore:
            wait_store(b, c - _NBUF)
        fire_gather(b)
        if drain:
            db = (b - _GD) % _NBUF
            wait_gather(db)
            fire_store(db, c - _GD)
        if issue:
            issue_in(b, c + _NBUF)

    for c in range(_NBUF):
        issue_in(c, c)
    for c in range(_NBUF):
        stage(c, c, drain=(c >= _GD), wstore=False)

    blocks = (_NCHUNK - 2 * _NBUF) // _NBUF

    def block_body(i, carry):
        c = _NBUF * i
        for j in range(_NBUF):
            stage(c + j, j)
        return carry
    lax.fori_loop(1, 1 + blocks, block_body, 0)

    for c in range(_NBUF * (1 + blocks), _NCHUNK):
        stage(c, c % _NBUF, issue=(c <= _NCHUNK - 1 - _NBUF))

    for c in range(_NCHUNK - _GD, _NCHUNK):
        wait_gather(c % _NBUF)
        fire_store(c % _NBUF, c)
    for c in range(_NCHUNK - _NBUF, _NCHUNK):
        wait_store(c % _NBUF, c)


def _sc_gather(table, state, obs):
    mesh = plsc.VectorSubcoreMesh(core_axis_name="c", subcore_axis_name="s")
    buf = lambda shape, dt: [pltpu.VMEM(shape, dt) for _ in range(_NBUF)]
    return pl.kernel(
        _sc_gather_body,
        out_type=jax.ShapeDtypeStruct((_N_LOOKUPS,), jnp.float32),
        mesh=mesh,
        scratch_types=(
            buf((_C,), jnp.int32)
            + buf((_C,), jnp.int32)
            + buf((_C,), jnp.int32)
            + buf((_C,), jnp.float32)
            + [pltpu.SemaphoreType.DMA for _ in range(3 * _NBUF)]
        ),
    )(table, state, obs)


def kernel(log_em, state, obs):
    flat = log_em.reshape(-1)
    return _sc_gather(flat, state.astype(jnp.int32), obs.astype(jnp.int32))


# speedup vs baseline: 1.0006x; 1.0006x over previous
import jax
import jax.numpy as jnp
from jax import lax
from jax.experimental import pallas as pl
from jax.experimental.pallas import tpu as pltpu
from jax.experimental.pallas import tpu_sc as plsc

_N_STATES = 65
_N_OBVS = 100000
_N_LOOKUPS = 3276800

_NC = 2
_NS = 16
_NW = _NC * _NS
_LANES = 16

_C = 2048
_G = 2048
_PW = _N_LOOKUPS // _NW
_NCHUNK = _PW // _C
_NG = _N_LOOKUPS // _C


_NBUF = 6
_GD = _NBUF - 1


def _sc_gather_body(flat_hbm, state_hbm, obs_hbm, out_hbm, *refs):
    n = _NBUF
    sv = refs[0 * n:1 * n]
    ov = refs[1 * n:2 * n]
    ix = refs[2 * n:3 * n]
    ot = refs[3 * n:4 * n]
    isem = refs[4 * n:5 * n]
    gsem = refs[5 * n:6 * n]
    osem = refs[6 * n:7 * n]

    cid = lax.axis_index("c")
    sid = lax.axis_index("s")
    wid = sid * _NC + cid
    base = wid * _PW

    def in_copies(b, c):
        off = base + c * _C
        return (pltpu.make_async_copy(state_hbm.at[pl.ds(off, _C)], sv[b],
                                      isem[b]),
                pltpu.make_async_copy(obs_hbm.at[pl.ds(off, _C)], ov[b],
                                      isem[b]))

    def issue_in(b, c):
        for cp in in_copies(b, c):
            cp.start()

    def wait_in(b, c):
        for cp in in_copies(b, c):
            cp.wait()

    def compute(b):
        def vec_body(j, carry):
            st = sv[b][pl.ds(j * _LANES, _LANES)]
            ob = ov[b][pl.ds(j * _LANES, _LANES)]
            ix[b][pl.ds(j * _LANES, _LANES)] = st * _N_OBVS + ob
            return carry
        lax.fori_loop(0, _C // _LANES, vec_body, 0, unroll=4)

    def gather_copies(b):
        return [pltpu.make_async_copy(flat_hbm.at[ix[b].at[pl.ds(k * _G, _G)]],
                                      ot[b].at[pl.ds(k * _G, _G)], gsem[b])
                for k in range(_C // _G)]

    def fire_gather(b):
        for cp in gather_copies(b):
            cp.start()

    def wait_gather(b):
        for cp in gather_copies(b):
            cp.wait()

    def store_copy(b, c):
        return pltpu.make_async_copy(ot[b], out_hbm.at[pl.ds(base + c * _C, _C)],
                                     osem[b])

    def fire_store(b, c):
        store_copy(b, c).start()

    def wait_store(b, c):
        store_copy(b, c).wait()

    def stage(c, b, drain=True, wstore=True, issue=True):
        wait_in(b, c)
        compute(b)
        if wstore:
            wait_store(b, c - _NBUF)
        fire_gather(b)
        if drain:
            db = (b - _GD) % _NBUF
            wait_gather(db)
            fire_store(db, c - _GD)
        if issue:
            issue_in(b, c + _NBUF)

    for c in range(_NBUF):
        issue_in(c, c)
    for c in range(_NBUF):
        stage(c, c, drain=(c >= _GD), wstore=False)

    blocks = (_NCHUNK - 2 * _NBUF) // _NBUF

    def block_body(i, carry):
        c = _NBUF * i
        for j in range(_NBUF):
            stage(c + j, j)
        return carry
    lax.fori_loop(1, 1 + blocks, block_body, 0)

    for c in range(_NBUF * (1 + blocks), _NCHUNK):
        stage(c, c % _NBUF, issue=(c <= _NCHUNK - 1 - _NBUF))

    for c in range(_NCHUNK - _GD, _NCHUNK):
        wait_gather(c % _NBUF)
        fire_store(c % _NBUF, c)
    for c in range(_NCHUNK - _NBUF, _NCHUNK):
        wait_store(c % _NBUF, c)


def _sc_gather(table, state, obs):
    mesh = plsc.VectorSubcoreMesh(core_axis_name="c", subcore_axis_name="s")
    buf = lambda shape, dt: [pltpu.VMEM(shape, dt) for _ in range(_NBUF)]
    return pl.kernel(
        _sc_gather_body,
        out_type=jax.ShapeDtypeStruct((_N_LOOKUPS,), jnp.float32),
        mesh=mesh,
        scratch_types=(
            buf((_C,), jnp.int32)
            + buf((_C,), jnp.int32)
            + buf((_C,), jnp.int32)
            + buf((_C,), jnp.float32)
            + [pltpu.SemaphoreType.DMA for _ in range(3 * _NBUF)]
        ),
    )(table, state, obs)


def kernel(log_em, state, obs):
    flat = log_em.reshape(-1)
    return _sc_gather(flat, state.astype(jnp.int32), obs.astype(jnp.int32))
